# baseline (speedup 1.0000x reference)
import functools

import jax
import jax.numpy as jnp
from jax import lax
from jax.experimental import pallas as pl
from jax.experimental.pallas import tpu as pltpu
from jax.experimental.pallas import tpu_sc as plsc

BATCH = 4096
T = 200
D = 64
V = 1000000
NC, NS = 2, 16
NW = NC * NS
G = 128
L = 16
NK = G // L
VBLK = (V + 127) // 128


def _pack_kernel(tt_ref, out_ref):
    x = tt_ref[...]
    xt = x.T
    out_ref[...] = jnp.concatenate([xt, xt], axis=1)


def _emb_kernel(idx_hbm, table_hbm, out_hbm, idx_v, g_v, o_v,
                gsem0, gsem1, osem0, osem1):
    wid = lax.axis_index("s") * NC + lax.axis_index("c")
    b0 = wid * G

    pltpu.sync_copy(idx_hbm.at[:, pl.ds(b0, G)], idx_v)

    bvecs = [lax.iota(jnp.int32, L) + L * k for k in range(NK)]
    zero = lax.iota(jnp.int32, L) * 0

    def g_copy(t, b, sem):
        return pltpu.make_async_copy(
            table_hbm.at[idx_v.at[t]], g_v.at[b], sem)

    def o_copy(t, b, sem):
        return pltpu.make_async_copy(
            o_v.at[b], out_hbm.at[t, :, wid, :, :], sem)

    def transpose(b):
        for k in range(NK):
            for d in range(D):
                vals = plsc.load_gather(g_v.at[b], [bvecs[k], zero + d])
                o_v[b, d // 8, d % 8, pl.ds(L * k, L)] = vals

    g_copy(0, 0, gsem0).start()
    g_copy(1, 1, gsem1).start()

    g_copy(0, 0, gsem0).wait()
    transpose(0)
    g_copy(2, 0, gsem0).start()
    o_copy(0, 0, osem0).start()
    g_copy(1, 1, gsem1).wait()
    transpose(1)
    g_copy(3, 1, gsem1).start()
    o_copy(1, 1, osem1).start()

    def body(i, carry):
        t0 = 2 * i
        g_copy(t0, 0, gsem0).wait()
        o_copy(t0 - 2, 0, osem0).wait()
        transpose(0)
        g_copy(t0 + 2, 0, gsem0).start()
        o_copy(t0, 0, osem0).start()
        g_copy(t0 + 1, 1, gsem1).wait()
        o_copy(t0 - 1, 1, osem1).wait()
        transpose(1)
        g_copy(t0 + 3, 1, gsem1).start()
        o_copy(t0 + 1, 1, osem1).start()
        return carry

    lax.fori_loop(1, T // 2 - 1, body, 0)

    tl = T - 2
    g_copy(tl, 0, gsem0).wait()
    o_copy(tl - 2, 0, osem0).wait()
    transpose(0)
    o_copy(tl, 0, osem0).start()
    g_copy(tl + 1, 1, gsem1).wait()
    o_copy(tl - 1, 1, osem1).wait()
    transpose(1)
    o_copy(tl + 1, 1, osem1).start()
    o_copy(tl, 0, osem0).wait()
    o_copy(tl + 1, 1, osem1).wait()


def kernel(text, table):
    packed = pl.pallas_call(
        _pack_kernel,
        grid=(VBLK,),
        in_specs=[pl.BlockSpec((D, 128), lambda j: (0, j))],
        out_specs=pl.BlockSpec((128, 128), lambda j: (j, 0)),
        out_shape=jax.ShapeDtypeStruct((V, 128), jnp.float32),
    )(table.T)
    tbl = packed

    idx = text.T
    mesh = plsc.VectorSubcoreMesh(core_axis_name="c", subcore_axis_name="s")

    run = functools.partial(
        pl.kernel,
        out_type=jax.ShapeDtypeStruct((T, D // 8, NW, 8, G), jnp.float32),
        mesh=mesh,
        scratch_types=[
            pltpu.VMEM((T, G), jnp.int32),
            pltpu.VMEM((2, G, 128), jnp.float32),
            pltpu.VMEM((2, 8, 8, G), jnp.float32),
            pltpu.SemaphoreType.DMA,
            pltpu.SemaphoreType.DMA,
            pltpu.SemaphoreType.DMA,
            pltpu.SemaphoreType.DMA,
        ],
        compiler_params=pltpu.CompilerParams(
            use_tc_tiling_on_sc=False, needs_layout_passes=False),
    )(_emb_kernel)

    out5 = run(idx, tbl)
    out = out5.transpose(2, 4, 0, 1, 3).reshape(BATCH, T, D)
    return out


# speedup vs baseline: 3.7403x; 3.7403x over previous
import functools

import jax
import jax.numpy as jnp
from jax import lax
from jax.experimental import pallas as pl
from jax.experimental.pallas import tpu as pltpu
from jax.experimental.pallas import tpu_sc as plsc

BATCH = 4096
T = 200
D = 64
V = 1000000
NC, NS = 2, 16
NW = NC * NS
G = 128
L = 16
NK = G // L
PB = 2048
VBLK = (V + PB - 1) // PB


def _pack_kernel(tt_ref, out_ref):
    x = tt_ref[...]
    xt = x.T
    out_ref[...] = jnp.concatenate([xt, xt], axis=1)


def _emb_kernel(idx_hbm, table_hbm, out_hbm, idx_v, g_v, o_v,
                gsem0, gsem1, osem0, osem1):
    wid = lax.axis_index("s") * NC + lax.axis_index("c")
    b0 = wid * G

    pltpu.sync_copy(idx_hbm.at[:, pl.ds(b0, G)], idx_v)

    bvecs = [lax.iota(jnp.int32, L) + L * k for k in range(NK)]
    zero = lax.iota(jnp.int32, L) * 0

    def g_copy(t, b, sem):
        return pltpu.make_async_copy(
            table_hbm.at[idx_v.at[t]], g_v.at[b], sem)

    def o_copy(t, b, sem):
        return pltpu.make_async_copy(
            o_v.at[b], out_hbm.at[t, :, wid, :, :], sem)

    def transpose(b):
        for d in range(D):
            vals = [plsc.load_gather(g_v.at[b], [bvecs[k], zero + d])
                    for k in range(NK)]
            for k in range(NK):
                o_v[b, d // 8, d % 8, pl.ds(L * k, L)] = vals[k]

    g_copy(0, 0, gsem0).start()
    g_copy(1, 1, gsem1).start()

    g_copy(0, 0, gsem0).wait()
    transpose(0)
    g_copy(2, 0, gsem0).start()
    o_copy(0, 0, osem0).start()
    g_copy(1, 1, gsem1).wait()
    transpose(1)
    g_copy(3, 1, gsem1).start()
    o_copy(1, 1, osem1).start()

    def body(i, carry):
        t0 = 2 * i
        g_copy(t0, 0, gsem0).wait()
        o_copy(t0 - 2, 0, osem0).wait()
        transpose(0)
        g_copy(t0 + 2, 0, gsem0).start()
        o_copy(t0, 0, osem0).start()
        g_copy(t0 + 1, 1, gsem1).wait()
        o_copy(t0 - 1, 1, osem1).wait()
        transpose(1)
        g_copy(t0 + 3, 1, gsem1).start()
        o_copy(t0 + 1, 1, osem1).start()
        return carry

    lax.fori_loop(1, T // 2 - 1, body, 0)

    tl = T - 2
    g_copy(tl, 0, gsem0).wait()
    o_copy(tl - 2, 0, osem0).wait()
    transpose(0)
    o_copy(tl, 0, osem0).start()
    g_copy(tl + 1, 1, gsem1).wait()
    o_copy(tl - 1, 1, osem1).wait()
    transpose(1)
    o_copy(tl + 1, 1, osem1).start()
    o_copy(tl, 0, osem0).wait()
    o_copy(tl + 1, 1, osem1).wait()


def kernel(text, table):
    packed = pl.pallas_call(
        _pack_kernel,
        grid=(VBLK,),
        in_specs=[pl.BlockSpec((D, PB), lambda j: (0, j))],
        out_specs=pl.BlockSpec((PB, 128), lambda j: (j, 0)),
        out_shape=jax.ShapeDtypeStruct((V, 128), jnp.float32),
    )(table.T)
    tbl = packed

    idx = text.T
    mesh = plsc.VectorSubcoreMesh(core_axis_name="c", subcore_axis_name="s")

    run = functools.partial(
        pl.kernel,
        out_type=jax.ShapeDtypeStruct((T, D // 8, NW, 8, G), jnp.float32),
        mesh=mesh,
        scratch_types=[
            pltpu.VMEM((T, G), jnp.int32),
            pltpu.VMEM((2, G, 128), jnp.float32),
            pltpu.VMEM((2, 8, 8, G), jnp.float32),
            pltpu.SemaphoreType.DMA,
            pltpu.SemaphoreType.DMA,
            pltpu.SemaphoreType.DMA,
            pltpu.SemaphoreType.DMA,
        ],
        compiler_params=pltpu.CompilerParams(
            use_tc_tiling_on_sc=False, needs_layout_passes=False),
    )(_emb_kernel)

    out5 = run(idx, tbl)
    out = out5.transpose(2, 4, 0, 1, 3).reshape(BATCH, T, D)
    return out


# speedup vs baseline: 3.8657x; 1.0335x over previous
import functools

import jax
import jax.numpy as jnp
from jax import lax
from jax.experimental import pallas as pl
from jax.experimental.pallas import tpu as pltpu
from jax.experimental.pallas import tpu_sc as plsc

BATCH = 4096
T = 200
D = 64
V = 1000000
NC, NS = 2, 16
NW = NC * NS
G = 128
L = 16
NK = G // L
PB = 2048
VBLK = (V + PB - 1) // PB


def _pack_kernel(tt_ref, out_ref):
    x = tt_ref[...]
    xt = x.T
    out_ref[...] = jnp.concatenate([xt, xt], axis=1)


def _emb_kernel(idx_hbm, table_hbm, out_hbm, idx_v, g_v, o_v,
                gsem0, gsem1, osem0, osem1):
    wid = lax.axis_index("s") * NC + lax.axis_index("c")
    b0 = wid * G

    pltpu.sync_copy(idx_hbm.at[:, pl.ds(b0, G)], idx_v)

    bvecs = [lax.iota(jnp.int32, L) + L * k for k in range(NK)]
    zero = lax.iota(jnp.int32, L) * 0

    def g_copy(t, b, sem):
        return pltpu.make_async_copy(
            table_hbm.at[idx_v.at[t]], g_v.at[b], sem)

    def o_copy(t, b, sem):
        return pltpu.make_async_copy(
            o_v.at[b], out_hbm.at[t, :, wid, :, :], sem)

    def transpose(b):
        @plsc.parallel_loop(0, D, 1, unroll=8)
        def _(d):
            vals = [plsc.load_gather(g_v.at[b], [bvecs[k], zero + d])
                    for k in range(NK)]
            for k in range(NK):
                o_v[b, d // 8, d % 8, pl.ds(L * k, L)] = vals[k]

    g_copy(0, 0, gsem0).start()
    g_copy(1, 1, gsem1).start()

    g_copy(0, 0, gsem0).wait()
    transpose(0)
    g_copy(2, 0, gsem0).start()
    o_copy(0, 0, osem0).start()
    g_copy(1, 1, gsem1).wait()
    transpose(1)
    g_copy(3, 1, gsem1).start()
    o_copy(1, 1, osem1).start()

    def body(i, carry):
        t0 = 2 * i
        g_copy(t0, 0, gsem0).wait()
        o_copy(t0 - 2, 0, osem0).wait()
        transpose(0)
        g_copy(t0 + 2, 0, gsem0).start()
        o_copy(t0, 0, osem0).start()
        g_copy(t0 + 1, 1, gsem1).wait()
        o_copy(t0 - 1, 1, osem1).wait()
        transpose(1)
        g_copy(t0 + 3, 1, gsem1).start()
        o_copy(t0 + 1, 1, osem1).start()
        return carry

    lax.fori_loop(1, T // 2 - 1, body, 0)

    tl = T - 2
    g_copy(tl, 0, gsem0).wait()
    o_copy(tl - 2, 0, osem0).wait()
    transpose(0)
    o_copy(tl, 0, osem0).start()
    g_copy(tl + 1, 1, gsem1).wait()
    o_copy(tl - 1, 1, osem1).wait()
    transpose(1)
    o_copy(tl + 1, 1, osem1).start()
    o_copy(tl, 0, osem0).wait()
    o_copy(tl + 1, 1, osem1).wait()


def kernel(text, table):
    packed = pl.pallas_call(
        _pack_kernel,
        grid=(VBLK,),
        in_specs=[pl.BlockSpec((D, PB), lambda j: (0, j))],
        out_specs=pl.BlockSpec((PB, 128), lambda j: (j, 0)),
        out_shape=jax.ShapeDtypeStruct((V, 128), jnp.float32),
    )(table.T)
    tbl = packed

    idx = text.T
    mesh = plsc.VectorSubcoreMesh(core_axis_name="c", subcore_axis_name="s")

    run = functools.partial(
        pl.kernel,
        out_type=jax.ShapeDtypeStruct((T, D // 8, NW, 8, G), jnp.float32),
        mesh=mesh,
        scratch_types=[
            pltpu.VMEM((T, G), jnp.int32),
            pltpu.VMEM((2, G, 128), jnp.float32),
            pltpu.VMEM((2, 8, 8, G), jnp.float32),
            pltpu.SemaphoreType.DMA,
            pltpu.SemaphoreType.DMA,
            pltpu.SemaphoreType.DMA,
            pltpu.SemaphoreType.DMA,
        ],
        compiler_params=pltpu.CompilerParams(
            use_tc_tiling_on_sc=False, needs_layout_passes=False),
    )(_emb_kernel)

    out5 = run(idx, tbl)
    out = out5.transpose(2, 4, 0, 1, 3).reshape(BATCH, T, D)
    return out


# speedup vs baseline: 4.7439x; 1.2272x over previous
import functools

import jax
import jax.numpy as jnp
from jax import lax
from jax.experimental import pallas as pl
from jax.experimental.pallas import tpu as pltpu
from jax.experimental.pallas import tpu_sc as plsc

BATCH = 4096
T = 200
D = 64
NC, NS = 2, 16
NW = NC * NS
G = 128
CG = 5
N_CHUNKS = T // CG


def _emb_kernel(idx_hbm, table_hbm, out_hbm, idx_v, rows_v,
                gsem0, gsem1, ssem0, ssem1):
    wid = lax.axis_index("s") * NC + lax.axis_index("c")
    b0 = wid * G

    def load_idx(c, b):
        pltpu.sync_copy(idx_hbm.at[pl.ds(c * CG, CG), pl.ds(b0, G)],
                        idx_v.at[b])

    def g_copies(b, sem):
        return [
            pltpu.make_async_copy(
                table_hbm.at[idx_v.at[b, j]],
                rows_v.at[b, j],
                sem,
            )
            for j in range(CG)
        ]

    def s_copy(c, b, sem):
        return pltpu.make_async_copy(
            rows_v.at[b],
            out_hbm.at[pl.ds(c * CG, CG), pl.ds(b0, G), :],
            sem,
        )

    def fire_gather(c, b, sem):
        load_idx(c, b)
        for cp in g_copies(b, sem):
            cp.start()

    def wait_gather(b, sem):
        for cp in g_copies(b, sem):
            cp.wait()

    fire_gather(0, 0, gsem0)
    fire_gather(1, 1, gsem1)

    def body(i, carry):
        c0 = 2 * i
        wait_gather(0, gsem0)
        s_copy(c0, 0, ssem0).start()
        wait_gather(1, gsem1)
        s_copy(c0 + 1, 1, ssem1).start()
        s_copy(c0, 0, ssem0).wait()
        fire_gather(c0 + 2, 0, gsem0)
        s_copy(c0 + 1, 1, ssem1).wait()
        fire_gather(c0 + 3, 1, gsem1)
        return carry

    lax.fori_loop(0, N_CHUNKS // 2 - 1, body, 0)

    cl = N_CHUNKS - 2
    wait_gather(0, gsem0)
    s_copy(cl, 0, ssem0).start()
    wait_gather(1, gsem1)
    s_copy(cl + 1, 1, ssem1).start()
    s_copy(cl, 0, ssem0).wait()
    s_copy(cl + 1, 1, ssem1).wait()


def kernel(text, table):
    idx = text.T
    mesh = plsc.VectorSubcoreMesh(core_axis_name="c", subcore_axis_name="s")

    run = functools.partial(
        pl.kernel,
        out_type=jax.ShapeDtypeStruct((T, BATCH, D), jnp.float32),
        mesh=mesh,
        scratch_types=[
            pltpu.VMEM((2, CG, G), jnp.int32),
            pltpu.VMEM((2, CG, G, D), jnp.float32),
            pltpu.SemaphoreType.DMA,
            pltpu.SemaphoreType.DMA,
            pltpu.SemaphoreType.DMA,
            pltpu.SemaphoreType.DMA,
        ],
        compiler_params=pltpu.CompilerParams(use_tc_tiling_on_sc=False),
    )(_emb_kernel)

    out = run(idx, table)
    return out.transpose(1, 0, 2)
